# baseline (speedup 1.0000x reference)
import jax
import jax.numpy as jnp
from jax.experimental import pallas as pl
from jax.experimental.pallas import tpu as pltpu
from jax.experimental.pallas import tpu_sc as plsc

FF = 2048
E = 8
TOPK = 2
BLK = 512
BM_ROUTER = 1024
SC_W = 16


def _router_body(x_ref, wr_ref, logits_ref):
    x = x_ref[...].astype(jnp.bfloat16)
    w = wr_ref[...].astype(jnp.bfloat16)
    logits_ref[...] = jax.lax.dot_general(
        x, w, (((1,), (1,)), ((), ())), preferred_element_type=jnp.float32)


def _moe_body(be_ref, x_ref, win_ref, wout_ref, out_ref):
    x = x_ref[...]
    win = win_ref[0]
    h = jax.lax.dot_general(
        x, win, (((1,), (1,)), ((), ())), preferred_element_type=jnp.float32)
    g = h[:, :FF]
    u = h[:, FF:]
    a = (g * jax.nn.sigmoid(g) * u).astype(jnp.bfloat16)
    wout = wout_ref[0]
    out_ref[...] = jax.lax.dot_general(
        a, wout, (((1,), (1,)), ((), ())), preferred_element_type=jnp.float32)


def _sc_dispatch(x, pos_even, pos_odd, P):
    T, d = x.shape
    mesh = plsc.VectorSubcoreMesh(core_axis_name="core", subcore_axis_name="subcore")

    @pl.kernel(out_type=jax.ShapeDtypeStruct((P, d), x.dtype), mesh=mesh)
    def dispatch_kernel(x_hbm, ie_hbm, io_hbm, o_hbm):
        def body(x_vmem, ie_vmem, io_vmem):
            pltpu.sync_copy(x_vmem, o_hbm.at[ie_vmem.at[0]])
            pltpu.sync_copy(x_vmem, o_hbm.at[io_vmem.at[0]])

        pltpu.emit_pipeline(
            body,
            grid=(T // SC_W,),
            in_specs=[
                pl.BlockSpec((SC_W, d), lambda i: (i, 0)),
                pl.BlockSpec((1, SC_W), lambda i: (0, i)),
                pl.BlockSpec((1, SC_W), lambda i: (0, i)),
            ],
            out_specs=[],
            core_axis_name=("core", "subcore"),
            dimension_semantics=(pltpu.PARALLEL,),
        )(x_hbm, ie_hbm, io_hbm)

    return dispatch_kernel(x, pos_even.reshape(1, T), pos_odd.reshape(1, T))


def kernel(layer_input, w_router, w_in, w_out):
    bsz, length, d = layer_input.shape
    T = bsz * length
    S = T * TOPK
    P = S + E * BLK
    NB = P // BLK
    x = layer_input.reshape(T, d)

    logits = pl.pallas_call(
        _router_body,
        grid=(T // BM_ROUTER,),
        in_specs=[
            pl.BlockSpec((BM_ROUTER, d), lambda i: (i, 0)),
            pl.BlockSpec((E, d), lambda i: (0, 0)),
        ],
        out_specs=pl.BlockSpec((BM_ROUTER, E), lambda i: (i, 0)),
        out_shape=jax.ShapeDtypeStruct((T, E), jnp.float32),
    )(x, w_router)

    top_vals, top_idx = jax.lax.top_k(logits, TOPK)
    gates = jax.nn.softmax(top_vals, axis=1)
    flat_e = top_idx.reshape(-1)
    onehot = (flat_e[:, None] == jnp.arange(E)[None, :]).astype(jnp.int32)
    csum = jnp.cumsum(onehot, axis=0)
    counts = csum[-1]
    rank = jnp.take_along_axis(csum, flat_e[:, None], axis=1)[:, 0] - 1
    padded_counts = ((counts + BLK - 1) // BLK) * BLK
    cum_pad = jnp.cumsum(padded_counts)
    pad_offset = cum_pad - padded_counts
    pos = pad_offset[flat_e] + rank
    starts = jnp.arange(NB, dtype=jnp.int32) * BLK
    block_expert = jnp.minimum(
        jnp.sum(starts[:, None] >= cum_pad[None, :], axis=1), E - 1
    ).astype(jnp.int32)

    pos2 = pos.reshape(T, TOPK)
    perm_tok = jnp.zeros((P,), jnp.int32).at[pos].set(
        jnp.arange(S, dtype=jnp.int32) // TOPK, unique_indices=True)
    x_b = x.astype(jnp.bfloat16)
    x_sorted = jnp.take(x_b, perm_tok, axis=0)

    w_in_b = w_in.astype(jnp.bfloat16)
    w_out_b = w_out.astype(jnp.bfloat16)
    grid_spec = pltpu.PrefetchScalarGridSpec(
        num_scalar_prefetch=1,
        grid=(NB,),
        in_specs=[
            pl.BlockSpec((BLK, d), lambda b, be: (b, 0)),
            pl.BlockSpec((1, 2 * FF, d), lambda b, be: (be[b], 0, 0)),
            pl.BlockSpec((1, d, FF), lambda b, be: (be[b], 0, 0)),
        ],
        out_specs=pl.BlockSpec((BLK, d), lambda b, be: (b, 0)),
    )
    y = pl.pallas_call(
        _moe_body,
        grid_spec=grid_spec,
        out_shape=jax.ShapeDtypeStruct((P, d), jnp.float32),
    )(block_expert, x_sorted, w_in_b, w_out_b)

    y0 = y[pos2[:, 0]]
    y1 = y[pos2[:, 1]]
    out = gates[:, 0:1] * y0 + gates[:, 1:2] * y1
    return out.reshape(bsz, length, d), logits


# speedup vs baseline: 1.0748x; 1.0748x over previous
import jax
import jax.numpy as jnp
from jax.experimental import pallas as pl
from jax.experimental.pallas import tpu as pltpu
from jax.experimental.pallas import tpu_sc as plsc

FF = 2048
E = 8
TOPK = 2
BLK = 512
BM_ROUTER = 1024
SC_W = 128


def _router_body(x_ref, wr_ref, logits_ref):
    x = x_ref[...].astype(jnp.bfloat16)
    w = wr_ref[...].astype(jnp.bfloat16)
    logits_ref[...] = jax.lax.dot_general(
        x, w, (((1,), (1,)), ((), ())), preferred_element_type=jnp.float32)


def _moe_body(be_ref, x_ref, win_ref, wout_ref, out_ref):
    x = x_ref[...].astype(jnp.bfloat16)
    win = win_ref[0]
    h = jax.lax.dot_general(
        x, win, (((1,), (1,)), ((), ())), preferred_element_type=jnp.float32)
    g = h[:, :FF]
    u = h[:, FF:]
    a = (g * jax.nn.sigmoid(g) * u).astype(jnp.bfloat16)
    wout = wout_ref[0]
    out_ref[...] = jax.lax.dot_general(
        a, wout, (((1,), (1,)), ((), ())), preferred_element_type=jnp.float32)


def _sc_dispatch(x_sub, pos_even, pos_odd, P):
    n_sub, dsub = x_sub.shape
    mesh = plsc.VectorSubcoreMesh(core_axis_name="core", subcore_axis_name="subcore")

    @pl.kernel(out_type=jax.ShapeDtypeStruct((P * 8, dsub), x_sub.dtype), mesh=mesh)
    def dispatch_kernel(x_hbm, ie_hbm, io_hbm, o_hbm):
        def body(x_vmem, ie_vmem, io_vmem):
            pltpu.sync_copy(x_vmem, o_hbm.at[ie_vmem.at[0]])
            pltpu.sync_copy(x_vmem, o_hbm.at[io_vmem.at[0]])

        pltpu.emit_pipeline(
            body,
            grid=(n_sub // SC_W,),
            in_specs=[
                pl.BlockSpec((SC_W, dsub), lambda i: (i, 0)),
                pl.BlockSpec((1, SC_W), lambda i: (0, i)),
                pl.BlockSpec((1, SC_W), lambda i: (0, i)),
            ],
            out_specs=[],
            core_axis_name=("core", "subcore"),
            dimension_semantics=(pltpu.PARALLEL,),
        )(x_hbm, ie_hbm, io_hbm)

    return dispatch_kernel(x_sub, pos_even.reshape(1, n_sub), pos_odd.reshape(1, n_sub))


def kernel(layer_input, w_router, w_in, w_out):
    bsz, length, d = layer_input.shape
    T = bsz * length
    S = T * TOPK
    P = S + E * BLK
    NB = P // BLK
    x = layer_input.reshape(T, d)

    logits = pl.pallas_call(
        _router_body,
        grid=(T // BM_ROUTER,),
        in_specs=[
            pl.BlockSpec((BM_ROUTER, d), lambda i: (i, 0)),
            pl.BlockSpec((E, d), lambda i: (0, 0)),
        ],
        out_specs=pl.BlockSpec((BM_ROUTER, E), lambda i: (i, 0)),
        out_shape=jax.ShapeDtypeStruct((T, E), jnp.float32),
    )(x, w_router)

    top_vals, top_idx = jax.lax.top_k(logits, TOPK)
    gates = jax.nn.softmax(top_vals, axis=1)
    flat_e = top_idx.reshape(-1)
    onehot = (flat_e[:, None] == jnp.arange(E)[None, :]).astype(jnp.int32)
    csum = jnp.cumsum(onehot, axis=0)
    counts = csum[-1]
    rank = jnp.take_along_axis(csum, flat_e[:, None], axis=1)[:, 0] - 1
    padded_counts = ((counts + BLK - 1) // BLK) * BLK
    cum_pad = jnp.cumsum(padded_counts)
    pad_offset = cum_pad - padded_counts
    pos = pad_offset[flat_e] + rank
    starts = jnp.arange(NB, dtype=jnp.int32) * BLK
    block_expert = jnp.minimum(
        jnp.sum(starts[:, None] >= cum_pad[None, :], axis=1), E - 1
    ).astype(jnp.int32)

    pos2 = pos.reshape(T, TOPK)
    sub = jnp.arange(8, dtype=jnp.int32)[None, :]
    pos_even_sub = (pos2[:, 0:1] * 8 + sub).reshape(-1)
    pos_odd_sub = (pos2[:, 1:2] * 8 + sub).reshape(-1)
    x_sub = x.reshape(T * 8, d // 8)
    x_sorted = _sc_dispatch(x_sub, pos_even_sub, pos_odd_sub, P).reshape(P, d)

    w_in_b = w_in.astype(jnp.bfloat16)
    w_out_b = w_out.astype(jnp.bfloat16)
    grid_spec = pltpu.PrefetchScalarGridSpec(
        num_scalar_prefetch=1,
        grid=(NB,),
        in_specs=[
            pl.BlockSpec((BLK, d), lambda b, be: (b, 0)),
            pl.BlockSpec((1, 2 * FF, d), lambda b, be: (be[b], 0, 0)),
            pl.BlockSpec((1, d, FF), lambda b, be: (be[b], 0, 0)),
        ],
        out_specs=pl.BlockSpec((BLK, d), lambda b, be: (b, 0)),
    )
    y = pl.pallas_call(
        _moe_body,
        grid_spec=grid_spec,
        out_shape=jax.ShapeDtypeStruct((P, d), jnp.float32),
    )(block_expert, x_sorted, w_in_b, w_out_b)

    y0 = y[pos2[:, 0]]
    y1 = y[pos2[:, 1]]
    out = gates[:, 0:1] * y0 + gates[:, 1:2] * y1
    return out.reshape(bsz, length, d), logits
